# baseline (speedup 1.0000x reference)
import functools

import jax
import jax.numpy as jnp
from jax import lax
from jax.experimental import pallas as pl
from jax.experimental.pallas import tpu as pltpu
from jax.experimental.pallas import tpu_sc as plsc

VOCAB = 21
D = 128
VP = 32
B = 4096
L = 500
NLANES = 16
NW = 32
ROWS_PER_W = B // NW
GROUPS = ROWS_PER_W // NLANES

_mesh = plsc.VectorSubcoreMesh(core_axis_name="c", subcore_axis_name="s")


@functools.partial(
    pl.kernel,
    out_type=jax.ShapeDtypeStruct((B * VP,), jnp.float32),
    mesh=_mesh,
    scratch_types=[
        pltpu.VMEM((ROWS_PER_W * L,), jnp.int32),
        pltpu.VMEM((ROWS_PER_W * VP,), jnp.float32),
    ],
    compiler_params=pltpu.CompilerParams(needs_layout_passes=False),
)
def _sc_hist(idx_hbm, cnt_hbm, idx_v, cnt_v):
    wid = lax.axis_index("s") * 2 + lax.axis_index("c")
    pltpu.sync_copy(idx_hbm.at[pl.ds(wid * (ROWS_PER_W * L), ROWS_PER_W * L)],
                    idx_v)

    zf = jnp.zeros((NLANES,), jnp.float32)

    @pl.loop(0, ROWS_PER_W * VP, step=NLANES)
    def _zero(i):
        cnt_v[pl.ds(i, NLANES)] = zf

    iota16 = lax.iota(jnp.int32, NLANES)
    ones = jnp.ones((NLANES,), jnp.float32)

    for g in range(GROUPS):
        av0 = iota16 * L + (g * NLANES * L)
        svec = iota16 * VP + (g * NLANES * VP)

        @pl.loop(0, L, init_carry=av0, unroll=10)
        def _acc(l, av, svec=svec):
            ids = plsc.load_gather(idx_v, [av])
            plsc.addupdate_scatter(cnt_v, [svec + ids], ones)
            return av + 1

    pltpu.sync_copy(cnt_v,
                    cnt_hbm.at[pl.ds(wid * (ROWS_PER_W * VP),
                                     ROWS_PER_W * VP)])


def _mm_body(cnt_ref, tab_ref, out_ref):
    out_ref[...] = lax.dot_general(
        cnt_ref[...], tab_ref[...],
        (((1,), (0,)), ((), ())),
        preferred_element_type=jnp.float32,
    ) * (1.0 / L)


_MM_BLK = 512


def _tc_matmul(counts, tablep):
    return pl.pallas_call(
        _mm_body,
        grid=(B // _MM_BLK,),
        in_specs=[
            pl.BlockSpec((_MM_BLK, VP), lambda i: (i, 0)),
            pl.BlockSpec((VP, D), lambda i: (0, 0)),
        ],
        out_specs=pl.BlockSpec((_MM_BLK, D), lambda i: (i, 0)),
        out_shape=jax.ShapeDtypeStruct((B, D), jnp.float32),
    )(counts, tablep)


def kernel(indices, table):
    indices = indices.astype(jnp.int32)
    table = table.astype(jnp.float32)
    counts = _sc_hist(indices.reshape(B * L)).reshape(B, VP)
    tablep = jnp.concatenate(
        [table, jnp.zeros((VP - VOCAB, D), jnp.float32)], axis=0)
    return _tc_matmul(counts, tablep)


# speedup vs baseline: 1.2665x; 1.2665x over previous
import functools

import jax
import jax.numpy as jnp
from jax import lax
from jax.experimental import pallas as pl
from jax.experimental.pallas import tpu as pltpu
from jax.experimental.pallas import tpu_sc as plsc

VOCAB = 21
D = 128
VP = 32
B = 4096
L = 500
NLANES = 16
NW = 32
ROWS_PER_W = B // NW
GROUPS = ROWS_PER_W // NLANES

_mesh = plsc.VectorSubcoreMesh(core_axis_name="c", subcore_axis_name="s")


@functools.partial(
    pl.kernel,
    out_type=jax.ShapeDtypeStruct((B, VP), jnp.float32),
    mesh=_mesh,
    scratch_types=[
        pltpu.VMEM((ROWS_PER_W, L), jnp.int32),
        pltpu.VMEM((ROWS_PER_W, VP), jnp.float32),
    ],
    compiler_params=pltpu.CompilerParams(needs_layout_passes=False),
)
def _sc_hist(idx_hbm, cnt_hbm, idx_v, cnt_v):
    wid = lax.axis_index("s") * 2 + lax.axis_index("c")
    base = wid * ROWS_PER_W
    pltpu.sync_copy(idx_hbm.at[pl.ds(base, ROWS_PER_W)], idx_v)

    zf = jnp.zeros((NLANES,), jnp.float32)

    @pl.loop(0, ROWS_PER_W)
    def _zero(r):
        cnt_v[r, pl.ds(0, NLANES)] = zf
        cnt_v[r, pl.ds(NLANES, NLANES)] = zf

    iota16 = lax.iota(jnp.int32, NLANES)
    ones = jnp.ones((NLANES,), jnp.float32)
    czero = jnp.zeros((NLANES,), jnp.int32)

    for g in range(GROUPS):
        rvec = iota16 + (g * NLANES)

        @plsc.parallel_loop(0, L, carry=czero, unroll=8)
        def _acc(l, cv, rvec=rvec):
            ids = plsc.load_gather(idx_v, [rvec, cv])
            plsc.addupdate_scatter(cnt_v, [rvec, ids], ones)
            return cv + 1

    pltpu.sync_copy(cnt_v, cnt_hbm.at[pl.ds(base, ROWS_PER_W)])


def _mm_body(cnt_ref, tab_ref, out_ref):
    out_ref[...] = lax.dot_general(
        cnt_ref[...], tab_ref[...],
        (((1,), (0,)), ((), ())),
        preferred_element_type=jnp.float32,
    ) * (1.0 / L)


_MM_BLK = 512


def _tc_matmul(counts, tablep):
    return pl.pallas_call(
        _mm_body,
        grid=(B // _MM_BLK,),
        in_specs=[
            pl.BlockSpec((_MM_BLK, VP), lambda i: (i, 0)),
            pl.BlockSpec((VP, D), lambda i: (0, 0)),
        ],
        out_specs=pl.BlockSpec((_MM_BLK, D), lambda i: (i, 0)),
        out_shape=jax.ShapeDtypeStruct((B, D), jnp.float32),
    )(counts, tablep)


def kernel(indices, table):
    indices = indices.astype(jnp.int32)
    table = table.astype(jnp.float32)
    counts = _sc_hist(indices)
    tablep = jnp.concatenate(
        [table, jnp.zeros((VP - VOCAB, D), jnp.float32)], axis=0)
    return _tc_matmul(counts, tablep)
